# baseline (speedup 1.0000x reference)
import functools

import jax
import jax.numpy as jnp
from jax import lax
from jax.experimental import pallas as pl
from jax.experimental.pallas import tpu as pltpu
from jax.experimental.pallas import tpu_sc as plsc

B = 4096
L = 200
EMB = 64
NC = 2
NS = 16
NW = NC * NS
RPW = B // NW
HALF = L // 2
CHUNK = 104
CPW = 2 * RPW


def _make_sc_pool():
    mesh = plsc.VectorSubcoreMesh(core_axis_name="c", subcore_axis_name="s")

    @functools.partial(
        pl.kernel,
        out_type=jax.ShapeDtypeStruct((B, EMB), jnp.float32),
        mesh=mesh,
        compiler_params=pltpu.CompilerParams(use_tc_tiling_on_sc=False),
        scratch_types=[
            pltpu.VMEM((CPW, CHUNK), jnp.int32),
            pltpu.VMEM((CHUNK, EMB), jnp.float32),
            pltpu.VMEM((CHUNK, EMB), jnp.float32),
            pltpu.VMEM((RPW, EMB), jnp.float32),
            pltpu.SemaphoreType.DMA,
            pltpu.SemaphoreType.DMA,
        ],
    )
    def sc_pool(idx_hbm, table_hbm, out_hbm, idx_v, buf_a, buf_b, pooled_v,
                sem_a, sem_b):
        cid = lax.axis_index("c")
        sid = lax.axis_index("s")
        wid = sid * NC + cid
        pltpu.sync_copy(idx_hbm.at[pl.ds(wid * CPW, CPW)], idx_v)
        zero = jnp.zeros((16,), jnp.float32)

        def row_body(r, carry):
            cp_a = pltpu.async_copy(table_hbm.at[idx_v.at[2 * r]], buf_a, sem_a)
            cp_b = pltpu.async_copy(table_hbm.at[idx_v.at[2 * r + 1]], buf_b,
                                    sem_b)
            cp_a.wait()
            cp_b.wait()

            def acc_body(i, accs):
                a = list(accs)
                for buf in (buf_a, buf_b):
                    for u in range(2):
                        row = 2 * i + u
                        for j in range(EMB // 16):
                            a[j] = a[j] + buf[row, pl.ds(16 * j, 16)]
                return tuple(a)

            accs = lax.fori_loop(0, HALF // 2, acc_body, (zero,) * (EMB // 16))
            for j in range(EMB // 16):
                pooled_v[r, pl.ds(16 * j, 16)] = accs[j]
            return carry

        lax.fori_loop(0, RPW, row_body, 0)
        pltpu.sync_copy(pooled_v, out_hbm.at[pl.ds(wid * RPW, RPW)])

    return sc_pool


_sc_pool = _make_sc_pool()


def _linear_body(p_ref, w_ref, b_ref, o_ref):
    pooled = p_ref[...] * (1.0 / L)
    acc = jnp.dot(pooled, w_ref[...].T, preferred_element_type=jnp.float32)
    o_ref[...] = jnp.maximum(acc + b_ref[...], 0.0)


def _linear(pooled_sum, w, b):
    return pl.pallas_call(
        _linear_body,
        out_shape=jax.ShapeDtypeStruct((B, EMB), jnp.float32),
    )(pooled_sum, w, b.reshape(1, EMB))


def kernel(x, table, W, b):
    xi = x.astype(jnp.int32).reshape(B * 2, HALF)
    idx = jnp.pad(xi, ((0, 0), (0, CHUNK - HALF)))
    pooled_sum = _sc_pool(idx, table)
    return _linear(pooled_sum, W, b)
